# initial kernel scaffold (unmeasured)
import jax
import jax.numpy as jnp
from jax import lax
from jax.experimental import pallas as pl
from jax.experimental.pallas import tpu as pltpu

N_DEV = 32


def kernel(x, w_mat):
    m_per, k = x.shape
    _, n_per = w_mat.shape

    def body(x_ref, w_ref, out_ref, comm_ref, send_sems, recv_sems):
        my = lax.axis_index("i")
        left = lax.rem(my - 1 + N_DEV, N_DEV)
        right = lax.rem(my + 1, N_DEV)

        barrier_sem = pltpu.get_barrier_semaphore()
        for nbr in (left, right):
            pl.semaphore_signal(
                barrier_sem, inc=1,
                device_id=(nbr,), device_id_type=pl.DeviceIdType.MESH,
            )
        pl.semaphore_wait(barrier_sem, 2)

        comm_ref[0] = x_ref[...]

        def gemm(slot):
            y = jnp.dot(
                comm_ref[slot], w_ref[...],
                preferred_element_type=jnp.float32,
            )
            y = y * jax.nn.sigmoid(y)
            origin = lax.rem(my - slot + N_DEV, N_DEV)
            out_ref[pl.ds(origin * m_per, m_per), :] = y

        for h in range(N_DEV - 1):
            rdma = pltpu.make_async_remote_copy(
                src_ref=comm_ref.at[h],
                dst_ref=comm_ref.at[h + 1],
                send_sem=send_sems.at[h],
                recv_sem=recv_sems.at[h],
                device_id=(right,),
                device_id_type=pl.DeviceIdType.MESH,
            )
            rdma.start()
            gemm(h)
            rdma.wait()
        gemm(N_DEV - 1)

    return pl.pallas_call(
        body,
        out_shape=jax.ShapeDtypeStruct((N_DEV * m_per, n_per), jnp.float32),
        in_specs=[
            pl.BlockSpec(memory_space=pltpu.VMEM),
            pl.BlockSpec(memory_space=pltpu.VMEM),
        ],
        out_specs=pl.BlockSpec(memory_space=pltpu.VMEM),
        scratch_shapes=[
            pltpu.VMEM((N_DEV, m_per, k), x.dtype),
            pltpu.SemaphoreType.DMA((N_DEV - 1,)),
            pltpu.SemaphoreType.DMA((N_DEV - 1,)),
        ],
        compiler_params=pltpu.CompilerParams(collective_id=0),
    )(x, w_mat)


# baseline (device time: 416767 ns/iter reference)
import jax
import jax.numpy as jnp
from jax import lax
from jax.experimental import pallas as pl
from jax.experimental.pallas import tpu as pltpu

N_DEV = 32


def kernel(x, w_mat):
    x = x.astype(jnp.bfloat16)
    w_mat = w_mat.astype(jnp.bfloat16)
    m_per, k = x.shape
    _, n_per = w_mat.shape

    def body(x_ref, w_ref, out_ref, comm_ref, send_sems, recv_sems):
        my = lax.axis_index("i")
        left = lax.rem(my - 1 + N_DEV, N_DEV)
        right = lax.rem(my + 1, N_DEV)

        barrier_sem = pltpu.get_barrier_semaphore()
        for nbr in (left, right):
            pl.semaphore_signal(
                barrier_sem, inc=1,
                device_id=(nbr,), device_id_type=pl.DeviceIdType.MESH,
            )
        pl.semaphore_wait(barrier_sem, 2)

        comm_ref[0] = x_ref[...]

        def gemm(slot):
            y = jnp.dot(
                comm_ref[slot], w_ref[...],
                preferred_element_type=jnp.float32,
            )
            y = y * jax.nn.sigmoid(y)
            origin = lax.rem(my - slot + N_DEV, N_DEV)
            out_ref[pl.ds(origin * m_per, m_per), :] = y

        for h in range(N_DEV - 1):
            rdma = pltpu.make_async_remote_copy(
                src_ref=comm_ref.at[h],
                dst_ref=comm_ref.at[h + 1],
                send_sem=send_sems.at[h],
                recv_sem=recv_sems.at[h],
                device_id=(right,),
                device_id_type=pl.DeviceIdType.MESH,
            )
            rdma.start()
            gemm(h)
            rdma.wait()
        gemm(N_DEV - 1)

    return pl.pallas_call(
        body,
        out_shape=jax.ShapeDtypeStruct((N_DEV * m_per, n_per), jnp.float32),
        in_specs=[
            pl.BlockSpec(memory_space=pltpu.VMEM),
            pl.BlockSpec(memory_space=pltpu.VMEM),
        ],
        out_specs=pl.BlockSpec(memory_space=pltpu.VMEM),
        scratch_shapes=[
            pltpu.VMEM((N_DEV, m_per, k), x.dtype),
            pltpu.SemaphoreType.DMA((N_DEV - 1,)),
            pltpu.SemaphoreType.DMA((N_DEV - 1,)),
        ],
        compiler_params=pltpu.CompilerParams(collective_id=0),
    )(x, w_mat)


# device time: 228737 ns/iter; 1.8220x vs baseline; 1.8220x over previous
import jax
import jax.numpy as jnp
from jax import lax
from jax.experimental import pallas as pl
from jax.experimental.pallas import tpu as pltpu

N_DEV = 32

_PLANE_SNAKE = [(0, 0), (1, 0), (1, 1), (0, 1), (0, 2), (1, 2), (1, 3), (0, 3)]
_LOGICAL_ORDER = [(x, y, z) for z in range(4) for x, y in _PLANE_SNAKE]
_COORD_TO_LOGICAL = {c: l for l, c in enumerate(_LOGICAL_ORDER)}

_C16 = [
    (0, 0), (0, 1), (0, 2), (0, 3),
    (1, 3), (1, 2), (1, 1),
    (2, 1), (2, 2), (2, 3),
    (3, 3), (3, 2), (3, 1), (3, 0),
    (2, 0), (1, 0),
]
_HAM = [(0, y, z) for y, z in _C16] + [(1, y, z) for y, z in reversed(_C16)]
assert len(set(_HAM)) == N_DEV
for _i in range(N_DEV):
    _a, _b = _HAM[_i], _HAM[(_i + 1) % N_DEV]
    assert sum(abs(p - q) for p, q in zip(_a, _b)) == 1, (_a, _b)

_PERM = [_COORD_TO_LOGICAL[c] for c in _HAM]
_RPOS = [0] * N_DEV
for _r, _l in enumerate(_PERM):
    _RPOS[_l] = _r

_N_R = 16
_N_L = 15

_RIGHT = [_PERM[(_RPOS[l] + 1) % N_DEV] for l in range(N_DEV)]
_LEFT = [_PERM[(_RPOS[l] - 1) % N_DEV] for l in range(N_DEV)]
_ORIG_R = [[_PERM[(_RPOS[l] - s) % N_DEV] for s in range(_N_R + 1)]
           for l in range(N_DEV)]
_ORIG_L = [[_PERM[(_RPOS[l] + s) % N_DEV] for s in range(_N_L + 1)]
           for l in range(N_DEV)]

_TBL = (
    [_RIGHT, _LEFT]
    + [[row[s] for row in _ORIG_R] for s in range(1, _N_R + 1)]
    + [[row[s] for row in _ORIG_L] for s in range(1, _N_L + 1)]
)


def kernel(x, w_mat):
    x = x.astype(jnp.bfloat16)
    w_mat = w_mat.astype(jnp.bfloat16)
    m_per, k = x.shape
    _, n_per = w_mat.shape

    def body(tbl_ref, x_ref, w_ref, out_ref, comm_r, comm_l,
             send_r_sems, recv_r_sems, send_l_sems, recv_l_sems):
        my = lax.axis_index("i")

        right = tbl_ref[0, my]
        left = tbl_ref[1, my]

        barrier_sem = pltpu.get_barrier_semaphore()
        for nbr in (left, right):
            pl.semaphore_signal(
                barrier_sem, inc=1,
                device_id=(nbr,), device_id_type=pl.DeviceIdType.MESH,
            )
        pl.semaphore_wait(barrier_sem, 2)

        def gemm(src_ref, origin):
            y = jnp.dot(src_ref[...], w_ref[...],
                        preferred_element_type=jnp.float32)
            y = y * jax.nn.sigmoid(y)
            out_ref[pl.ds(origin * m_per, m_per), :] = y

        for h in range(_N_R):
            rdma_r = pltpu.make_async_remote_copy(
                src_ref=x_ref if h == 0 else comm_r.at[h],
                dst_ref=comm_r.at[h + 1],
                send_sem=send_r_sems.at[h],
                recv_sem=recv_r_sems.at[h],
                device_id=(right,),
                device_id_type=pl.DeviceIdType.MESH,
            )
            rdma_r.start()
            rdma_l = None
            if h < _N_L:
                rdma_l = pltpu.make_async_remote_copy(
                    src_ref=x_ref if h == 0 else comm_l.at[h],
                    dst_ref=comm_l.at[h + 1],
                    send_sem=send_l_sems.at[h],
                    recv_sem=recv_l_sems.at[h],
                    device_id=(left,),
                    device_id_type=pl.DeviceIdType.MESH,
                )
                rdma_l.start()

            if h == 0:
                gemm(x_ref, my)
            else:
                gemm(comm_r.at[h], tbl_ref[2 + (h - 1), my])
                gemm(comm_l.at[h], tbl_ref[2 + _N_R + (h - 1), my])

            rdma_r.wait()
            if rdma_l is not None:
                rdma_l.wait()

        gemm(comm_r.at[_N_R], tbl_ref[2 + _N_R - 1, my])
        gemm(comm_l.at[_N_L], tbl_ref[2 + _N_R + _N_L - 1, my])

    tbl = jnp.asarray(_TBL, dtype=jnp.int32)

    return pl.pallas_call(
        body,
        out_shape=jax.ShapeDtypeStruct((N_DEV * m_per, n_per), jnp.float32),
        in_specs=[
            pl.BlockSpec(memory_space=pltpu.SMEM),
            pl.BlockSpec(memory_space=pltpu.VMEM),
            pl.BlockSpec(memory_space=pltpu.VMEM),
        ],
        out_specs=pl.BlockSpec(memory_space=pltpu.VMEM),
        scratch_shapes=[
            pltpu.VMEM((_N_R + 1, m_per, k), x.dtype),
            pltpu.VMEM((_N_L + 1, m_per, k), x.dtype),
            pltpu.SemaphoreType.DMA((_N_R,)),
            pltpu.SemaphoreType.DMA((_N_R,)),
            pltpu.SemaphoreType.DMA((_N_L,)),
            pltpu.SemaphoreType.DMA((_N_L,)),
        ],
        compiler_params=pltpu.CompilerParams(
            collective_id=0,
            vmem_limit_bytes=60 * 1024 * 1024,
        ),
    )(tbl, x, w_mat)


# device time: 201250 ns/iter; 2.0709x vs baseline; 1.1366x over previous
import jax
import jax.numpy as jnp
from jax import lax
from jax.experimental import pallas as pl
from jax.experimental.pallas import tpu as pltpu

N_DEV = 32

_PLANE_SNAKE = [(0, 0), (1, 0), (1, 1), (0, 1), (0, 2), (1, 2), (1, 3), (0, 3)]
_LOGICAL_ORDER = [(x, y, z) for z in range(4) for x, y in _PLANE_SNAKE]
_COORD_TO_LOGICAL = {c: l for l, c in enumerate(_LOGICAL_ORDER)}

_C16 = [
    (0, 0), (0, 1), (0, 2), (0, 3),
    (1, 3), (1, 2), (1, 1),
    (2, 1), (2, 2), (2, 3),
    (3, 3), (3, 2), (3, 1), (3, 0),
    (2, 0), (1, 0),
]
_HAM = [(0, y, z) for y, z in _C16] + [(1, y, z) for y, z in reversed(_C16)]
assert len(set(_HAM)) == N_DEV
for _i in range(N_DEV):
    _a, _b = _HAM[_i], _HAM[(_i + 1) % N_DEV]
    assert sum(abs(p - q) for p, q in zip(_a, _b)) == 1, (_a, _b)

_PERM = [_COORD_TO_LOGICAL[c] for c in _HAM]
_RPOS = [0] * N_DEV
for _r, _l in enumerate(_PERM):
    _RPOS[_l] = _r

_N_R = 16
_N_L = 15

_RIGHT = [_PERM[(_RPOS[l] + 1) % N_DEV] for l in range(N_DEV)]
_LEFT = [_PERM[(_RPOS[l] - 1) % N_DEV] for l in range(N_DEV)]
_ORIG_R = [[_PERM[(_RPOS[l] - s) % N_DEV] for s in range(_N_R + 1)]
           for l in range(N_DEV)]
_ORIG_L = [[_PERM[(_RPOS[l] + s) % N_DEV] for s in range(_N_L + 1)]
           for l in range(N_DEV)]

_TBL = (
    [_RIGHT, _LEFT]
    + [[row[s] for row in _ORIG_R] for s in range(1, _N_R + 1)]
    + [[row[s] for row in _ORIG_L] for s in range(1, _N_L + 1)]
)


def kernel(x, w_mat):
    x = x.astype(jnp.bfloat16)
    w_mat = w_mat.astype(jnp.bfloat16)
    m_per, k = x.shape
    _, n_per = w_mat.shape

    def body(tbl_ref, x_ref, w_ref, out_ref, comm_r, comm_l,
             send_r_sems, recv_r_sems, send_l_sems, recv_l_sems):
        my = lax.axis_index("i")

        right = tbl_ref[0, my]
        left = tbl_ref[1, my]

        barrier_sem = pltpu.get_barrier_semaphore()
        for nbr in (left, right):
            pl.semaphore_signal(
                barrier_sem, inc=1,
                device_id=(nbr,), device_id_type=pl.DeviceIdType.MESH,
            )
        pl.semaphore_wait(barrier_sem, 2)

        def gemm(src_ref, origin):
            y = jnp.dot(src_ref[...], w_ref[...],
                        preferred_element_type=jnp.float32)
            y = y * jax.nn.sigmoid(y)
            out_ref[pl.ds(origin * m_per, m_per), :] = y

        k_half = k // 2

        def mk(h, half, comm, send_sems, recv_sems, dev):
            lo = half * k_half
            src = (x_ref.at[:, pl.ds(lo, k_half)] if h == 0
                   else comm.at[h, :, pl.ds(lo, k_half)])
            return pltpu.make_async_remote_copy(
                src_ref=src,
                dst_ref=comm.at[h + 1, :, pl.ds(lo, k_half)],
                send_sem=send_sems.at[half, h],
                recv_sem=recv_sems.at[half, h],
                device_id=(dev,),
                device_id_type=pl.DeviceIdType.MESH,
            )

        def mk_r(h, half):
            return mk(h, half, comm_r, send_r_sems, recv_r_sems, right)

        def mk_l(h, half):
            return mk(h, half, comm_l, send_l_sems, recv_l_sems, left)

        started = []

        def start(d):
            d.start()
            started.append(d)

        for half in (0, 1):
            start(mk_r(0, half))
        for half in (0, 1):
            start(mk_l(0, half))
        gemm(x_ref, my)

        for h in range(1, _N_R + 1):
            mk_r(h - 1, 0).wait_recv()
            if h < _N_R:
                start(mk_r(h, 0))
            mk_r(h - 1, 1).wait_recv()
            if h < _N_R:
                start(mk_r(h, 1))
            if h <= _N_L:
                mk_l(h - 1, 0).wait_recv()
                if h < _N_L:
                    start(mk_l(h, 0))
                mk_l(h - 1, 1).wait_recv()
                if h < _N_L:
                    start(mk_l(h, 1))
            gemm(comm_r.at[h], tbl_ref[2 + (h - 1), my])
            if h <= _N_L:
                gemm(comm_l.at[h], tbl_ref[2 + _N_R + (h - 1), my])

        for d in started:
            d.wait_send()

    tbl = jnp.asarray(_TBL, dtype=jnp.int32)

    return pl.pallas_call(
        body,
        out_shape=jax.ShapeDtypeStruct((N_DEV * m_per, n_per), jnp.float32),
        in_specs=[
            pl.BlockSpec(memory_space=pltpu.SMEM),
            pl.BlockSpec(memory_space=pltpu.VMEM),
            pl.BlockSpec(memory_space=pltpu.VMEM),
        ],
        out_specs=pl.BlockSpec(memory_space=pltpu.VMEM),
        scratch_shapes=[
            pltpu.VMEM((_N_R + 1, m_per, k), x.dtype),
            pltpu.VMEM((_N_L + 1, m_per, k), x.dtype),
            pltpu.SemaphoreType.DMA((2, _N_R)),
            pltpu.SemaphoreType.DMA((2, _N_R)),
            pltpu.SemaphoreType.DMA((2, _N_L)),
            pltpu.SemaphoreType.DMA((2, _N_L)),
        ],
        compiler_params=pltpu.CompilerParams(
            collective_id=0,
            vmem_limit_bytes=60 * 1024 * 1024,
        ),
    )(tbl, x, w_mat)


# device time: 195776 ns/iter; 2.1288x vs baseline; 1.0280x over previous
import jax
import jax.numpy as jnp
from jax import lax
from jax.experimental import pallas as pl
from jax.experimental.pallas import tpu as pltpu

N_DEV = 32

_PLANE_SNAKE = [(0, 0), (1, 0), (1, 1), (0, 1), (0, 2), (1, 2), (1, 3), (0, 3)]
_LOGICAL_ORDER = [(x, y, z) for z in range(4) for x, y in _PLANE_SNAKE]
_COORD_TO_LOGICAL = {c: l for l, c in enumerate(_LOGICAL_ORDER)}

_C16 = [
    (0, 0), (0, 1), (0, 2), (0, 3),
    (1, 3), (1, 2), (1, 1),
    (2, 1), (2, 2), (2, 3),
    (3, 3), (3, 2), (3, 1), (3, 0),
    (2, 0), (1, 0),
]
_HAM = [(0, y, z) for y, z in _C16] + [(1, y, z) for y, z in reversed(_C16)]
assert len(set(_HAM)) == N_DEV
for _i in range(N_DEV):
    _a, _b = _HAM[_i], _HAM[(_i + 1) % N_DEV]
    assert sum(abs(p - q) for p, q in zip(_a, _b)) == 1, (_a, _b)

_PERM = [_COORD_TO_LOGICAL[c] for c in _HAM]
_RPOS = [0] * N_DEV
for _r, _l in enumerate(_PERM):
    _RPOS[_l] = _r

_N_R = 16
_N_L = 15

_RIGHT = [_PERM[(_RPOS[l] + 1) % N_DEV] for l in range(N_DEV)]
_LEFT = [_PERM[(_RPOS[l] - 1) % N_DEV] for l in range(N_DEV)]
_ORIG_R = [[_PERM[(_RPOS[l] - s) % N_DEV] for s in range(_N_R + 1)]
           for l in range(N_DEV)]
_ORIG_L = [[_PERM[(_RPOS[l] + s) % N_DEV] for s in range(_N_L + 1)]
           for l in range(N_DEV)]

_TBL = (
    [_RIGHT, _LEFT]
    + [[row[s] for row in _ORIG_R] for s in range(1, _N_R + 1)]
    + [[row[s] for row in _ORIG_L] for s in range(1, _N_L + 1)]
)


def kernel(x, w_mat):
    m_per, k = x.shape
    _, n_per = w_mat.shape

    def body(tbl_ref, x_ref, w_ref, out_ref, x_bf, w_bf, comm_r, comm_l,
             send_r_sems, recv_r_sems, send_l_sems, recv_l_sems):
        my = lax.axis_index("i")

        right = tbl_ref[0, my]
        left = tbl_ref[1, my]

        barrier_sem = pltpu.get_barrier_semaphore()
        for nbr in (left, right):
            pl.semaphore_signal(
                barrier_sem, inc=1,
                device_id=(nbr,), device_id_type=pl.DeviceIdType.MESH,
            )
        pl.semaphore_wait(barrier_sem, 2)

        def gemm(src_ref, origin):
            y = jnp.dot(src_ref[...], w_bf[...],
                        preferred_element_type=jnp.float32)
            y = y * jax.nn.sigmoid(y)
            out_ref[pl.ds(origin * m_per, m_per), :] = y

        k_half = k // 2

        def mk(h, half, comm, send_sems, recv_sems, dev):
            lo = half * k_half
            src = (x_bf.at[:, pl.ds(lo, k_half)] if h == 0
                   else comm.at[h, :, pl.ds(lo, k_half)])
            return pltpu.make_async_remote_copy(
                src_ref=src,
                dst_ref=comm.at[h + 1, :, pl.ds(lo, k_half)],
                send_sem=send_sems.at[half, h],
                recv_sem=recv_sems.at[half, h],
                device_id=(dev,),
                device_id_type=pl.DeviceIdType.MESH,
            )

        def mk_r(h, half):
            return mk(h, half, comm_r, send_r_sems, recv_r_sems, right)

        def mk_l(h, half):
            return mk(h, half, comm_l, send_l_sems, recv_l_sems, left)

        started = []

        def start(d):
            d.start()
            started.append(d)

        x_bf[...] = x_ref[...].astype(jnp.bfloat16)
        for half in (0, 1):
            start(mk_r(0, half))
        for half in (0, 1):
            start(mk_l(0, half))
        w_bf[...] = w_ref[...].astype(jnp.bfloat16)
        gemm(x_bf, my)

        for h in range(1, _N_R + 1):
            mk_r(h - 1, 0).wait_recv()
            if h < _N_R:
                start(mk_r(h, 0))
            mk_r(h - 1, 1).wait_recv()
            if h < _N_R:
                start(mk_r(h, 1))
            if h <= _N_L:
                mk_l(h - 1, 0).wait_recv()
                if h < _N_L:
                    start(mk_l(h, 0))
                mk_l(h - 1, 1).wait_recv()
                if h < _N_L:
                    start(mk_l(h, 1))
            gemm(comm_r.at[h], tbl_ref[2 + (h - 1), my])
            if h <= _N_L:
                gemm(comm_l.at[h], tbl_ref[2 + _N_R + (h - 1), my])

        for d in started:
            d.wait_send()

    tbl = jnp.asarray(_TBL, dtype=jnp.int32)

    return pl.pallas_call(
        body,
        out_shape=jax.ShapeDtypeStruct((N_DEV * m_per, n_per), jnp.float32),
        in_specs=[
            pl.BlockSpec(memory_space=pltpu.SMEM),
            pl.BlockSpec(memory_space=pltpu.VMEM),
            pl.BlockSpec(memory_space=pltpu.VMEM),
        ],
        out_specs=pl.BlockSpec(memory_space=pltpu.VMEM),
        scratch_shapes=[
            pltpu.VMEM((m_per, k), jnp.bfloat16),
            pltpu.VMEM((k, n_per), jnp.bfloat16),
            pltpu.VMEM((_N_R + 1, m_per, k), jnp.bfloat16),
            pltpu.VMEM((_N_L + 1, m_per, k), jnp.bfloat16),
            pltpu.SemaphoreType.DMA((2, _N_R)),
            pltpu.SemaphoreType.DMA((2, _N_R)),
            pltpu.SemaphoreType.DMA((2, _N_L)),
            pltpu.SemaphoreType.DMA((2, _N_L)),
        ],
        compiler_params=pltpu.CompilerParams(
            collective_id=0,
            vmem_limit_bytes=60 * 1024 * 1024,
        ),
    )(tbl, x, w_mat)


# device time: 148549 ns/iter; 2.8056x vs baseline; 1.3179x over previous
import jax
import jax.numpy as jnp
from jax import lax
from jax.experimental import pallas as pl
from jax.experimental.pallas import tpu as pltpu

N_DEV = 32

_PLANE_SNAKE = [(0, 0), (1, 0), (1, 1), (0, 1), (0, 2), (1, 2), (1, 3), (0, 3)]
_LOGICAL_ORDER = [(x, y, z) for z in range(4) for x, y in _PLANE_SNAKE]
_COORD_TO_LOGICAL = {c: l for l, c in enumerate(_LOGICAL_ORDER)}

_C16 = [
    (0, 0), (0, 1), (0, 2), (0, 3),
    (1, 3), (1, 2), (1, 1),
    (2, 1), (2, 2), (2, 3),
    (3, 3), (3, 2), (3, 1), (3, 0),
    (2, 0), (1, 0),
]
assert len(set(_C16)) == 16
for _i in range(16):
    _a, _b = _C16[_i], _C16[(_i + 1) % 16]
    assert abs(_a[0] - _b[0]) + abs(_a[1] - _b[1]) == 1, (_a, _b)

_NF = 8
_NL = 7
_MX = 80
_MR = 48
_MP = 176
_SUBS = [(0, 96), (96, 32), (128, 48)]


def _lid(x, yz):
    return _COORD_TO_LOGICAL[(x, yz[0], yz[1])]


_R16 = [0] * N_DEV
_L16 = [0] * N_DEV
_PART = [0] * N_DEV
_OF = [[0] * N_DEV for _ in range(_NF + 1)]
_OL = [[0] * N_DEV for _ in range(_NL + 1)]
_POF = [[0] * N_DEV for _ in range(_NF + 1)]
_POL = [[0] * N_DEV for _ in range(_NL + 1)]
for _l, (_x, _y, _z) in enumerate(_LOGICAL_ORDER):
    _q = _C16.index((_y, _z))
    _R16[_l] = _lid(_x, _C16[(_q + 1) % 16])
    _L16[_l] = _lid(_x, _C16[(_q - 1) % 16])
    _PART[_l] = _lid(1 - _x, (_y, _z))
    for _h in range(1, _NF + 1):
        _OF[_h][_l] = _lid(_x, _C16[(_q - _h) % 16])
        _POF[_h][_l] = _lid(1 - _x, _C16[(_q - _h) % 16])
    for _h in range(1, _NL + 1):
        _OL[_h][_l] = _lid(_x, _C16[(_q + _h) % 16])
        _POL[_h][_l] = _lid(1 - _x, _C16[(_q + _h) % 16])

_TBL = (
    [_R16, _L16, _PART]
    + [_OF[h] for h in range(1, _NF + 1)]
    + [_OL[h] for h in range(1, _NL + 1)]
    + [_POF[h] for h in range(1, _NF + 1)]
    + [_POL[h] for h in range(1, _NL + 1)]
)
_ROW_OF = lambda h: 3 + (h - 1)
_ROW_OL = lambda h: 3 + _NF + (h - 1)
_ROW_POF = lambda h: 3 + _NF + _NL + (h - 1)
_ROW_POL = lambda h: 3 + 2 * _NF + _NL + (h - 1)


def kernel(x, w_mat):
    m_per, k = x.shape
    _, n_per = w_mat.shape

    def body(tbl_ref, x_ref, w_ref, out_ref,
             x_bf, px, w_bf, comm_f, comm_l, xin_f, xin_l,
             sf_send, sf_recv, sl_send, sl_recv, sx_send, sx_recv):
        my = lax.axis_index("i")
        right = tbl_ref[0, my]
        left = tbl_ref[1, my]
        partner = tbl_ref[2, my]

        barrier_sem = pltpu.get_barrier_semaphore()
        for nbr in (left, right, partner):
            pl.semaphore_signal(
                barrier_sem, inc=1,
                device_id=(nbr,), device_id_type=pl.DeviceIdType.MESH,
            )
        pl.semaphore_wait(barrier_sem, 3)

        started = []

        def start(d):
            d.start()
            started.append(d)

        def rdma(src, dst, ssem, rsem, dev):
            return pltpu.make_async_remote_copy(
                src_ref=src, dst_ref=dst, send_sem=ssem, recv_sem=rsem,
                device_id=(dev,), device_id_type=pl.DeviceIdType.MESH,
            )

        def mk_f(h, m):
            r0, rn = _SUBS[m]
            if h == 0:
                src = (x_bf.at[pl.ds(0, 96)], x_bf.at[pl.ds(96, 32)],
                       px.at[pl.ds(80, 48)])[m]
            else:
                src = comm_f.at[h, pl.ds(r0, rn)]
            return rdma(src, comm_f.at[h + 1, pl.ds(r0, rn)],
                        sf_send.at[m, h], sf_recv.at[m, h], right)

        def mk_l(h, m):
            r0, rn = _SUBS[m]
            if h == 0:
                src = (x_bf.at[pl.ds(0, 96)], x_bf.at[pl.ds(96, 32)],
                       px.at[pl.ds(80, 48)])[m]
            else:
                src = comm_l.at[h, pl.ds(r0, rn)]
            return rdma(src, comm_l.at[h + 1, pl.ds(r0, rn)],
                        sl_send.at[m, h], sl_recv.at[m, h], left)

        def mk_x_p1():
            return rdma(x_bf.at[pl.ds(80, 48)], px.at[pl.ds(80, 48)],
                        sx_send.at[0], sx_recv.at[0], partner)

        def mk_x_p2():
            return rdma(x_bf.at[pl.ds(0, 80)], px.at[pl.ds(0, 80)],
                        sx_send.at[1], sx_recv.at[1], partner)

        def mk_x_f(h):
            return rdma(comm_f.at[h, pl.ds(0, _MX)], xin_f.at[h - 1],
                        sx_send.at[2 + (h - 1)], sx_recv.at[2 + (h - 1)],
                        partner)

        def mk_x_l(h):
            return rdma(comm_l.at[h, pl.ds(0, _MX)], xin_l.at[h - 1],
                        sx_send.at[2 + _NF + (h - 1)],
                        sx_recv.at[2 + _NF + (h - 1)], partner)

        def silu_store(y, origin, row0, nrows):
            out_ref[pl.ds(origin * m_per + row0, nrows), :] = (
                y * jax.nn.sigmoid(y))

        def gemm_payload(slot_ref, owner, powner):
            y = jnp.dot(slot_ref[...], w_bf[...],
                        preferred_element_type=jnp.float32)
            silu_store(y[0:m_per], owner, 0, m_per)
            silu_store(y[m_per:_MP], powner, _MX, _MR)

        def gemm_full(src_ref, origin):
            y = jnp.dot(src_ref[...], w_bf[...],
                        preferred_element_type=jnp.float32)
            silu_store(y, origin, 0, m_per)

        def gemm_x(xin_ref, powner):
            y = jnp.dot(xin_ref[...], w_bf[...],
                        preferred_element_type=jnp.float32)
            silu_store(y, powner, 0, _MX)

        x_bf[...] = x_ref[...].astype(jnp.bfloat16)
        start(mk_x_p1())
        start(mk_x_p2())
        for m in (0, 1):
            start(mk_f(0, m))
            start(mk_l(0, m))
        w_bf[...] = w_ref[...].astype(jnp.bfloat16)
        mk_x_p1().wait_recv()
        start(mk_f(0, 2))
        start(mk_l(0, 2))
        gemm_full(x_bf, my)
        mk_x_p2().wait_recv()
        gemm_full(px, partner)

        for s in range(1, _NF + 1):
            for m in range(3):
                mk_f(s - 1, m).wait_recv()
                if s < _NF:
                    start(mk_f(s, m))
            start(mk_x_f(s))
            if s <= _NL:
                for m in range(3):
                    mk_l(s - 1, m).wait_recv()
                    if s < _NL:
                        start(mk_l(s, m))
                start(mk_x_l(s))
            gemm_payload(comm_f.at[s], tbl_ref[_ROW_OF(s), my],
                         tbl_ref[_ROW_POF(s), my])
            if s <= _NL:
                gemm_payload(comm_l.at[s], tbl_ref[_ROW_OL(s), my],
                             tbl_ref[_ROW_POL(s), my])
            if s >= 2:
                mk_x_f(s - 1).wait_recv()
                gemm_x(xin_f.at[s - 2], tbl_ref[_ROW_POF(s - 1), my])
                mk_x_l(s - 1).wait_recv()
                gemm_x(xin_l.at[s - 2], tbl_ref[_ROW_POL(s - 1), my])

        mk_x_f(_NF).wait_recv()
        gemm_x(xin_f.at[_NF - 1], tbl_ref[_ROW_POF(_NF), my])

        for d in started:
            d.wait_send()

    tbl = jnp.asarray(_TBL, dtype=jnp.int32)

    return pl.pallas_call(
        body,
        out_shape=jax.ShapeDtypeStruct((N_DEV * m_per, n_per), jnp.float32),
        in_specs=[
            pl.BlockSpec(memory_space=pltpu.SMEM),
            pl.BlockSpec(memory_space=pltpu.VMEM),
            pl.BlockSpec(memory_space=pltpu.VMEM),
        ],
        out_specs=pl.BlockSpec(memory_space=pltpu.VMEM),
        scratch_shapes=[
            pltpu.VMEM((m_per, k), jnp.bfloat16),
            pltpu.VMEM((m_per, k), jnp.bfloat16),
            pltpu.VMEM((k, n_per), jnp.bfloat16),
            pltpu.VMEM((_NF + 1, _MP, k), jnp.bfloat16),
            pltpu.VMEM((_NL + 1, _MP, k), jnp.bfloat16),
            pltpu.VMEM((_NF, _MX, k), jnp.bfloat16),
            pltpu.VMEM((_NL, _MX, k), jnp.bfloat16),
            pltpu.SemaphoreType.DMA((3, _NF)),
            pltpu.SemaphoreType.DMA((3, _NF)),
            pltpu.SemaphoreType.DMA((3, _NL)),
            pltpu.SemaphoreType.DMA((3, _NL)),
            pltpu.SemaphoreType.DMA((2 + _NF + _NL,)),
            pltpu.SemaphoreType.DMA((2 + _NF + _NL,)),
        ],
        compiler_params=pltpu.CompilerParams(
            collective_id=0,
            vmem_limit_bytes=60 * 1024 * 1024,
        ),
    )(tbl, x, w_mat)


# device time: 141880 ns/iter; 2.9375x vs baseline; 1.0470x over previous
import jax
import jax.numpy as jnp
from jax import lax
from jax.experimental import pallas as pl
from jax.experimental.pallas import tpu as pltpu

N_DEV = 32

_PLANE_SNAKE = [(0, 0), (1, 0), (1, 1), (0, 1), (0, 2), (1, 2), (1, 3), (0, 3)]
_LOGICAL_ORDER = [(x, y, z) for z in range(4) for x, y in _PLANE_SNAKE]
_COORD_TO_LOGICAL = {c: l for l, c in enumerate(_LOGICAL_ORDER)}

_C16 = [
    (0, 0), (0, 1), (0, 2), (0, 3),
    (1, 3), (1, 2), (1, 1),
    (2, 1), (2, 2), (2, 3),
    (3, 3), (3, 2), (3, 1), (3, 0),
    (2, 0), (1, 0),
]
assert len(set(_C16)) == 16
for _i in range(16):
    _a, _b = _C16[_i], _C16[(_i + 1) % 16]
    assert abs(_a[0] - _b[0]) + abs(_a[1] - _b[1]) == 1, (_a, _b)

_NF = 8
_NL = 7
_MX = 80
_MR = 48
_MP = 176
_SUBS = [(0, 80), (80, 48), (128, 48)]


def _lid(x, yz):
    return _COORD_TO_LOGICAL[(x, yz[0], yz[1])]


_R16 = [0] * N_DEV
_L16 = [0] * N_DEV
_PART = [0] * N_DEV
_OF = [[0] * N_DEV for _ in range(_NF + 1)]
_OL = [[0] * N_DEV for _ in range(_NL + 1)]
_POF = [[0] * N_DEV for _ in range(_NF + 1)]
_POL = [[0] * N_DEV for _ in range(_NL + 1)]
for _l, (_x, _y, _z) in enumerate(_LOGICAL_ORDER):
    _q = _C16.index((_y, _z))
    _R16[_l] = _lid(_x, _C16[(_q + 1) % 16])
    _L16[_l] = _lid(_x, _C16[(_q - 1) % 16])
    _PART[_l] = _lid(1 - _x, (_y, _z))
    for _h in range(1, _NF + 1):
        _OF[_h][_l] = _lid(_x, _C16[(_q - _h) % 16])
        _POF[_h][_l] = _lid(1 - _x, _C16[(_q - _h) % 16])
    for _h in range(1, _NL + 1):
        _OL[_h][_l] = _lid(_x, _C16[(_q + _h) % 16])
        _POL[_h][_l] = _lid(1 - _x, _C16[(_q + _h) % 16])

_TBL = (
    [_R16, _L16, _PART]
    + [_OF[h] for h in range(1, _NF + 1)]
    + [_OL[h] for h in range(1, _NL + 1)]
    + [_POF[h] for h in range(1, _NF + 1)]
    + [_POL[h] for h in range(1, _NL + 1)]
)
_ROW_OF = lambda h: 3 + (h - 1)
_ROW_OL = lambda h: 3 + _NF + (h - 1)
_ROW_POF = lambda h: 3 + _NF + _NL + (h - 1)
_ROW_POL = lambda h: 3 + 2 * _NF + _NL + (h - 1)


def kernel(x, w_mat):
    m_per, k = x.shape
    _, n_per = w_mat.shape

    def body(tbl_ref, x_ref, w_ref, out_ref,
             x_bf, px, w_bf, comm_f, comm_l, xin_f, xin_l,
             sf_send, sf_recv, sl_send, sl_recv, sx_send, sx_recv):
        my = lax.axis_index("i")
        right = tbl_ref[0, my]
        left = tbl_ref[1, my]
        partner = tbl_ref[2, my]

        barrier_sem = pltpu.get_barrier_semaphore()
        for nbr in (left, right, partner):
            pl.semaphore_signal(
                barrier_sem, inc=1,
                device_id=(nbr,), device_id_type=pl.DeviceIdType.MESH,
            )
        pl.semaphore_wait(barrier_sem, 3)

        started = []

        def start(d):
            d.start()
            started.append(d)

        def rdma(src, dst, ssem, rsem, dev):
            return pltpu.make_async_remote_copy(
                src_ref=src, dst_ref=dst, send_sem=ssem, recv_sem=rsem,
                device_id=(dev,), device_id_type=pl.DeviceIdType.MESH,
            )

        def mk_f(h, m):
            r0, rn = _SUBS[m]
            if h == 0:
                src = (x_bf.at[pl.ds(0, 80)], x_bf.at[pl.ds(80, 48)],
                       px.at[pl.ds(80, 48)])[m]
            else:
                src = comm_f.at[h, pl.ds(r0, rn)]
            return rdma(src, comm_f.at[h + 1, pl.ds(r0, rn)],
                        sf_send.at[m, h], sf_recv.at[m, h], right)

        def mk_l(h, m):
            r0, rn = _SUBS[m]
            if h == 0:
                src = (x_bf.at[pl.ds(0, 80)], x_bf.at[pl.ds(80, 48)],
                       px.at[pl.ds(80, 48)])[m]
            else:
                src = comm_l.at[h, pl.ds(r0, rn)]
            return rdma(src, comm_l.at[h + 1, pl.ds(r0, rn)],
                        sl_send.at[m, h], sl_recv.at[m, h], left)

        def mk_x_p1():
            return rdma(x_bf.at[pl.ds(80, 48)], px.at[pl.ds(80, 48)],
                        sx_send.at[0], sx_recv.at[0], partner)

        def mk_x_p2():
            return rdma(x_bf.at[pl.ds(0, 80)], px.at[pl.ds(0, 80)],
                        sx_send.at[1], sx_recv.at[1], partner)

        def mk_x_f(h):
            return rdma(comm_f.at[h, pl.ds(0, _MX)], xin_f.at[h - 1],
                        sx_send.at[2 + (h - 1)], sx_recv.at[2 + (h - 1)],
                        partner)

        def mk_x_l(h):
            return rdma(comm_l.at[h, pl.ds(0, _MX)], xin_l.at[h - 1],
                        sx_send.at[2 + _NF + (h - 1)],
                        sx_recv.at[2 + _NF + (h - 1)], partner)

        def silu_store(y, origin, row0, nrows):
            out_ref[pl.ds(origin * m_per + row0, nrows), :] = (
                y * jax.nn.sigmoid(y))

        def gemm_payload(slot_ref, owner, powner):
            y = jnp.dot(slot_ref[...], w_bf[...],
                        preferred_element_type=jnp.float32)
            silu_store(y[0:m_per], owner, 0, m_per)
            silu_store(y[m_per:_MP], powner, _MX, _MR)

        def gemm_full(src_ref, origin):
            y = jnp.dot(src_ref[...], w_bf[...],
                        preferred_element_type=jnp.float32)
            silu_store(y, origin, 0, m_per)

        def gemm_x(xin_ref, powner):
            y = jnp.dot(xin_ref[...], w_bf[...],
                        preferred_element_type=jnp.float32)
            silu_store(y, powner, 0, _MX)

        x_bf[...] = x_ref[...].astype(jnp.bfloat16)
        start(mk_x_p1())
        start(mk_x_p2())
        for m in (0, 1):
            start(mk_f(0, m))
            start(mk_l(0, m))
        w_bf[...] = w_ref[...].astype(jnp.bfloat16)
        mk_x_p1().wait_recv()
        start(mk_f(0, 2))
        start(mk_l(0, 2))
        gemm_full(x_bf, my)
        mk_x_p2().wait_recv()
        gemm_full(px, partner)

        for s in range(1, _NF + 1):
            for m in range(3):
                mk_f(s - 1, m).wait_recv()
                if s < _NF:
                    start(mk_f(s, m))
                if m == 0:
                    start(mk_x_f(s))
            if s <= _NL:
                for m in range(3):
                    mk_l(s - 1, m).wait_recv()
                    if s < _NL:
                        start(mk_l(s, m))
                    if m == 0:
                        start(mk_x_l(s))
            gemm_payload(comm_f.at[s], tbl_ref[_ROW_OF(s), my],
                         tbl_ref[_ROW_POF(s), my])
            if s <= _NL:
                gemm_payload(comm_l.at[s], tbl_ref[_ROW_OL(s), my],
                             tbl_ref[_ROW_POL(s), my])
            if s >= 2:
                mk_x_f(s - 1).wait_recv()
                gemm_x(xin_f.at[s - 2], tbl_ref[_ROW_POF(s - 1), my])
                mk_x_l(s - 1).wait_recv()
                gemm_x(xin_l.at[s - 2], tbl_ref[_ROW_POL(s - 1), my])

        mk_x_f(_NF).wait_recv()
        gemm_x(xin_f.at[_NF - 1], tbl_ref[_ROW_POF(_NF), my])

        for d in started:
            d.wait_send()

    tbl = jnp.asarray(_TBL, dtype=jnp.int32)

    return pl.pallas_call(
        body,
        out_shape=jax.ShapeDtypeStruct((N_DEV * m_per, n_per), jnp.float32),
        in_specs=[
            pl.BlockSpec(memory_space=pltpu.SMEM),
            pl.BlockSpec(memory_space=pltpu.VMEM),
            pl.BlockSpec(memory_space=pltpu.VMEM),
        ],
        out_specs=pl.BlockSpec(memory_space=pltpu.VMEM),
        scratch_shapes=[
            pltpu.VMEM((m_per, k), jnp.bfloat16),
            pltpu.VMEM((m_per, k), jnp.bfloat16),
            pltpu.VMEM((k, n_per), jnp.bfloat16),
            pltpu.VMEM((_NF + 1, _MP, k), jnp.bfloat16),
            pltpu.VMEM((_NL + 1, _MP, k), jnp.bfloat16),
            pltpu.VMEM((_NF, _MX, k), jnp.bfloat16),
            pltpu.VMEM((_NL, _MX, k), jnp.bfloat16),
            pltpu.SemaphoreType.DMA((3, _NF)),
            pltpu.SemaphoreType.DMA((3, _NF)),
            pltpu.SemaphoreType.DMA((3, _NL)),
            pltpu.SemaphoreType.DMA((3, _NL)),
            pltpu.SemaphoreType.DMA((2 + _NF + _NL,)),
            pltpu.SemaphoreType.DMA((2 + _NF + _NL,)),
        ],
        compiler_params=pltpu.CompilerParams(
            collective_id=0,
            vmem_limit_bytes=60 * 1024 * 1024,
        ),
    )(tbl, x, w_mat)
